# baseline (speedup 1.0000x reference)
import functools

import jax
import jax.numpy as jnp
from jax.experimental import pallas as pl

_SCALE = 0.044194173824159216
_TEMPERATURE = 0.2
_C = -_SCALE / _TEMPERATURE

_BN = 2048


def _dist_kernel(q_ref, k_ref, o_ref):
    q = q_ref[...].astype(jnp.bfloat16)
    k = k_ref[...].astype(jnp.bfloat16)
    acc = jax.lax.dot_general(
        q, k, (((1,), (1,)), ((), ())),
        preferred_element_type=jnp.float32)
    o_ref[...] = acc * _C


@functools.partial(jax.jit, static_argnames=())
def kernel(query, key):
    m, d = query.shape
    n = key.shape[0]
    grid = (n // _BN,)
    return pl.pallas_call(
        _dist_kernel,
        grid=grid,
        in_specs=[
            pl.BlockSpec((m, d), lambda i: (0, 0)),
            pl.BlockSpec((_BN, d), lambda i: (i, 0)),
        ],
        out_specs=pl.BlockSpec((m, _BN), lambda i: (0, i)),
        out_shape=jax.ShapeDtypeStruct((m, n), jnp.float32),
    )(query, key)


# speedup vs baseline: 1.0437x; 1.0437x over previous
import functools

import jax
import jax.numpy as jnp
from jax.experimental import pallas as pl

_SCALE = 0.044194173824159216
_TEMPERATURE = 0.2
_C = -_SCALE / _TEMPERATURE

_BN = 4096


def _dist_kernel(q_ref, k_ref, o_ref):
    q = q_ref[...].astype(jnp.bfloat16)
    k = k_ref[...].astype(jnp.bfloat16)
    acc = jax.lax.dot_general(
        q, k, (((1,), (1,)), ((), ())),
        preferred_element_type=jnp.float32)
    o_ref[...] = acc * _C


@functools.partial(jax.jit, static_argnames=())
def kernel(query, key):
    m, d = query.shape
    n = key.shape[0]
    grid = (n // _BN,)
    return pl.pallas_call(
        _dist_kernel,
        grid=grid,
        in_specs=[
            pl.BlockSpec((m, d), lambda i: (0, 0)),
            pl.BlockSpec((_BN, d), lambda i: (i, 0)),
        ],
        out_specs=pl.BlockSpec((m, _BN), lambda i: (0, i)),
        out_shape=jax.ShapeDtypeStruct((m, n), jnp.float32),
    )(query, key)
